# baseline (speedup 1.0000x reference)
import jax
import jax.numpy as jnp
from jax import lax
from jax.experimental import pallas as pl
from jax.experimental.pallas import tpu as pltpu
from jax.experimental.pallas import tpu_sc as plsc

N = 10000
E = 160000
D = 128
ALPHA = 0.2

NW = 32
NPW = 320
NPAD = NW * NPW
EPW = E // NW
BCAP = 288
ABUF = NW * BCAP
CAP = 5888
SZ = CAP + 64
BR = 64
NEG = -3.4e38
HUGE = 0x7FFFFFFF


def _tc_body(x_ref, w_ref, a2_ref, wh_ref, st_ref, cs_ref):
    i = pl.program_id(0)
    wh = jnp.dot(x_ref[...], w_ref[...], preferred_element_type=jnp.float32)
    wh_ref[...] = wh
    st_ref[...] = jnp.dot(wh, a2_ref[...], preferred_element_type=jnp.float32)

    @pl.when(i == 0)
    def _():
        cs_ref[...] = jnp.zeros_like(cs_ref)

    cs_ref[...] += jnp.sum(wh, axis=0, keepdims=True)


def _tc_call(X, W, A2):
    return pl.pallas_call(
        _tc_body,
        grid=(10,),
        in_specs=[
            pl.BlockSpec((1000, D), lambda i: (i, 0)),
            pl.BlockSpec((D, D), lambda i: (0, 0)),
            pl.BlockSpec((D, 2), lambda i: (0, 0)),
        ],
        out_specs=[
            pl.BlockSpec((1000, D), lambda i: (i, 0)),
            pl.BlockSpec((1000, 2), lambda i: (i, 0)),
            pl.BlockSpec((1, D), lambda i: (0, 0)),
        ],
        out_shape=[
            jax.ShapeDtypeStruct((N, D), jnp.float32),
            jax.ShapeDtypeStruct((N, 2), jnp.float32),
            jax.ShapeDtypeStruct((1, D), jnp.float32),
        ],
    )(X, W, A2)


def _pa_body(src_hbm, dst_hbm, abuf_hbm, acnt_hbm, src_v, dst_v, ab_v, ac_v):
    wid = lax.axis_index("c") * 16 + lax.axis_index("s")
    iota = lax.iota(jnp.int32, 16)
    zi = jnp.zeros((16,), jnp.int32)

    pltpu.sync_copy(src_hbm.at[pl.ds(wid * EPW, EPW)], src_v.at[pl.ds(0, EPW)])
    pltpu.sync_copy(dst_hbm.at[pl.ds(wid * EPW, EPW)], dst_v.at[pl.ds(0, EPW)])

    for q in range(3):
        ac_v[pl.ds(q * 16, 16)] = zi

    def _scan(j, _):
        sv = src_v[pl.ds(j * 16, 16)]
        dv = dst_v[pl.ds(j * 16, 16)]
        valid = j * 16 + iota < EPW
        key = jnp.where(valid, sv * 16384 + dv, jnp.int32(HUGE))
        sk, _si = plsc.sort_key_val(key, iota)
        vs = sk < HUGE
        own = jnp.where(vs, ((sk >> 14) * 6554) >> 21, jnp.int32(32))
        prev = own[jnp.clip(iota - 1, 0, 15)]
        neq = (own != prev) | (iota == 0)
        start = plsc.cummax(jnp.where(neq, iota, 0))
        rank = iota - start
        base = plsc.load_gather(ac_v, [own])
        slot = jnp.minimum(base + rank, BCAP - 1)
        pos = jnp.minimum(own, NW - 1) * BCAP + slot
        plsc.store_scatter(ab_v, [pos], sk, mask=vs)
        nxt = own[jnp.clip(iota + 1, 0, 15)]
        is_last = ((own != nxt) | (iota == 15)) & vs
        plsc.store_scatter(ac_v, [own], jnp.minimum(slot + 1, BCAP),
                           mask=is_last)
        return 0

    lax.fori_loop(0, (EPW + 15) // 16, _scan, 0)

    pltpu.sync_copy(ab_v, abuf_hbm.at[pl.ds(wid * ABUF, ABUF)])
    pltpu.sync_copy(ac_v.at[pl.ds(0, 32)], acnt_hbm.at[pl.ds(wid * 32, 32)])


def _pa_call(src, dst):
    mesh = plsc.VectorSubcoreMesh(core_axis_name="c", subcore_axis_name="s")
    f = pl.kernel(
        _pa_body,
        out_type=[
            jax.ShapeDtypeStruct((NW * ABUF,), jnp.int32),
            jax.ShapeDtypeStruct((NW * 32,), jnp.int32),
        ],
        mesh=mesh,
        compiler_params=pltpu.CompilerParams(needs_layout_passes=False),
        scratch_types=[
            pltpu.VMEM((EPW + 16,), jnp.int32),
            pltpu.VMEM((EPW + 16,), jnp.int32),
            pltpu.VMEM((ABUF,), jnp.int32),
            pltpu.VMEM((48,), jnp.int32),
        ],
    )
    return f(src, dst)


def _pb_body(abuf_hbm, acnt_hbm, s_hbm, t_hbm, wh_hbm, cs_hbm, out_hbm,
             s_v, t_v, cs_v, ac_all, bb_v, src_s, dst_s, e_s, att_s,
             hist, rp, cur, m_arr, rz_arr, stamp, dstb, gbuf, out_v, sem):
    wid = lax.axis_index("c") * 16 + lax.axis_index("s")
    lo = wid * NPW
    iota = lax.iota(jnp.int32, 16)
    zf = jnp.zeros((16,), jnp.float32)
    zi = jnp.zeros((16,), jnp.int32)

    def spl(x):
        return jnp.full((16,), x)

    def g1(ref, i):
        return plsc.load_gather(ref, [spl(i)])

    def s1(ref, i, x):
        xv = x if getattr(x, "shape", ()) == (16,) else spl(x)
        plsc.store_scatter(ref, [spl(i)], xv)

    pltpu.sync_copy(s_hbm, s_v)
    pltpu.sync_copy(t_hbm, t_v)
    pltpu.sync_copy(cs_hbm, cs_v)
    pltpu.sync_copy(acnt_hbm, ac_all)

    def _ld(s2, _):
        pltpu.sync_copy(abuf_hbm.at[pl.ds(s2 * ABUF + wid * BCAP, BCAP)],
                        bb_v.at[pl.ds(s2 * BCAP, BCAP)])
        return 0
    lax.fori_loop(0, NW, _ld, 0)

    def _mz(j, _):
        hist[pl.ds(j * 16, 16)] = zi
        return 0
    lax.fori_loop(0, (NPW + 16) // 16, _mz, 0)

    def _ms(j, _):
        stamp[pl.ds(j * 16, 16)] = zi - 1
        return 0
    lax.fori_loop(0, N // 16, _ms, 0)

    def _md(j, _):
        dst_s[pl.ds(j * 16, 16)] = zi
        return 0
    lax.fori_loop(0, SZ // 16, _md, 0)

    def _mo(i, _):
        for c in range(8):
            out_v[i, pl.ds(c * 16, 16)] = zf
        return 0
    lax.fori_loop(0, NPW, _mo, 0)

    def _h_seg(s2, _):
        cnt_s = g1(ac_all, s2 * 32 + wid)[0]

        def _h(j, _):
            kv = bb_v[pl.ds(s2 * BCAP + j * 16, 16)]
            key = jnp.where(j * 16 + iota < cnt_s, kv, jnp.int32(HUGE))
            sk, _si = plsc.sort_key_val(key, iota)
            vs = sk < HUGE
            row = jnp.where(vs, (sk >> 14) - lo, jnp.int32(NPW))
            prev = row[jnp.clip(iota - 1, 0, 15)]
            neq = (row != prev) | (iota == 0)
            start = plsc.cummax(jnp.where(neq, iota, 0))
            rank = iota - start
            base = plsc.load_gather(hist, [row])
            nxt = row[jnp.clip(iota + 1, 0, 15)]
            is_last = (row != nxt) | (iota == 15)
            plsc.store_scatter(hist, [row], base + rank + 1, mask=is_last)
            return 0

        lax.fori_loop(0, (cnt_s + 15) // 16, _h, 0)
        return 0
    lax.fori_loop(0, NW, _h_seg, 0)

    def _p(k, acc):
        v = hist[pl.ds(k * 16, 16)]
        c = plsc.cumsum(v)
        rp[pl.ds(k * 16, 16)] = spl(acc) + c - v
        cur[pl.ds(k * 16, 16)] = spl(acc) + c - v
        return acc + c[15]
    cnt = lax.fori_loop(0, NPW // 16, _p, jnp.int32(0))
    s1(rp, NPW, cnt)

    def _pl_seg(s2, _):
        cnt_s = g1(ac_all, s2 * 32 + wid)[0]

        def _pl2(j, _):
            kv = bb_v[pl.ds(s2 * BCAP + j * 16, 16)]
            key = jnp.where(j * 16 + iota < cnt_s, kv, jnp.int32(HUGE))
            sk, _si = plsc.sort_key_val(key, iota)
            vs = sk < HUGE
            row = jnp.where(vs, (sk >> 14) - lo, jnp.int32(NPW))
            prev = row[jnp.clip(iota - 1, 0, 15)]
            neq = (row != prev) | (iota == 0)
            start = plsc.cummax(jnp.where(neq, iota, 0))
            rank = iota - start
            base = plsc.load_gather(cur, [row])
            pos = jnp.clip(base + rank, 0, SZ - 1)
            plsc.store_scatter(src_s, [pos], sk >> 14, mask=vs)
            plsc.store_scatter(dst_s, [pos], sk & 16383, mask=vs)
            nxt = row[jnp.clip(iota + 1, 0, 15)]
            is_last = (row != nxt) | (iota == 15)
            plsc.store_scatter(cur, [row], base + rank + 1, mask=is_last)
            return 0

        lax.fori_loop(0, (cnt_s + 15) // 16, _pl2, 0)
        return 0
    lax.fori_loop(0, NW, _pl_seg, 0)

    nv = ((cnt + BR - 1) // BR) * (BR // 16)

    def _e(j, _):
        idxc = jnp.minimum(j * 16 + iota, cnt - 1)
        sv = plsc.load_gather(src_s, [idxc])
        dv = plsc.load_gather(dst_s, [idxc])
        z = plsc.load_gather(s_v, [sv]) + plsc.load_gather(t_v, [dv])
        e_s[pl.ds(j * 16, 16)] = jnp.where(z > 0, z, ALPHA * z)
        return 0
    lax.fori_loop(0, nv, _e, 0)

    negv = jnp.full((16,), NEG, jnp.float32)

    def _d(j, _):
        idxv = j * 16 + iota
        idxc = jnp.minimum(idxv, cnt - 1)
        valid = idxv < cnt
        sv = plsc.load_gather(src_s, [idxc])
        dv = plsc.load_gather(dst_s, [idxc])
        sl = jnp.clip(sv - lo, 0, NPW - 1)
        key2 = jnp.where(valid, dv * 512 + sl, jnp.int32(HUGE))
        sk, si = plsc.sort_key_val(key2, iota)
        gidx = j * 16 + si
        prev = sk[jnp.clip(iota - 1, 0, 15)]
        eqprev = (sk == prev) & (iota > 0) & (sk < HUGE)
        plsc.store_scatter(e_s, [jnp.clip(gidx, 0, SZ - 1)], negv,
                           mask=eqprev)
        rp0 = plsc.load_gather(rp, [sl])
        old = plsc.load_gather(stamp, [dv])
        dupb = valid & (old >= rp0)
        plsc.store_scatter(e_s, [idxc], negv, mask=dupb)
        nxt = sk[jnp.clip(iota + 1, 0, 15)]
        is_last = (((sk >> 9) != (nxt >> 9)) | (iota == 15)) & (sk < HUGE)
        plsc.store_scatter(stamp, [jnp.clip(sk >> 9, 0, N - 1)], gidx,
                           mask=is_last)
        return 0
    lax.fori_loop(0, (cnt + 15) // 16, _d, 0)

    def _r(i, _):
        r01 = plsc.load_gather(rp, [i + iota])
        r0 = r01[0]
        r1 = r01[1]

        @pl.when(r1 > r0)
        def _():
            nch = (r1 - r0 + 15) // 16

            def _m(c2, m):
                idxv = r0 + c2 * 16 + iota
                ev = plsc.load_gather(e_s, [jnp.minimum(idxv, r1 - 1)])
                return jnp.maximum(m, jnp.max(jnp.where(idxv < r1, ev, NEG)))

            m = lax.fori_loop(0, nch, _m, jnp.float32(NEG))

            def _z(c2, za):
                idxv = r0 + c2 * 16 + iota
                ev = plsc.load_gather(e_s, [jnp.minimum(idxv, r1 - 1)])
                return za + jnp.sum(jnp.where(idxv < r1, jnp.exp(ev - m), 0.0))

            zs = lax.fori_loop(0, nch, _z, jnp.float32(0.0))
            s1(m_arr, i, m)
            s1(rz_arr, i, 1.0 / spl(zs))
        return 0
    lax.fori_loop(0, NPW, _r, 0)

    def _a(j, _):
        idxv = j * 16 + iota
        idxc = jnp.minimum(idxv, cnt - 1)
        sv = plsc.load_gather(src_s, [idxc])
        sl = jnp.clip(sv - lo, 0, NPW - 1)
        mrow = plsc.load_gather(m_arr, [sl])
        rz = plsc.load_gather(rz_arr, [sl])
        ev = plsc.load_gather(e_s, [idxc])
        att = jnp.where(idxv < cnt, jnp.exp(ev - mrow) * rz, 0.0)
        att_s[pl.ds(j * 16, 16)] = att
        return 0
    lax.fori_loop(0, nv, _a, 0)

    nb = (cnt + BR - 1) // BR

    def _b(b, _):
        base = b * BR
        for q in range(BR // 16):
            dstb[pl.ds(q * 16, 16)] = dst_s[pl.ds(base + q * 16, 16)]
        pltpu.async_copy(wh_hbm.at[dstb], gbuf, sem).wait()

        def _acc(l, _):
            r = base + l
            al = g1(att_s, r)
            sl = jnp.clip(g1(src_s, r)[0] - lo, 0, NPW - 1)
            for c in range(8):
                seg = pl.ds(c * 16, 16)
                out_v[sl, seg] = out_v[sl, seg] + al * gbuf[l, seg]
            return 0
        lax.fori_loop(0, BR, _acc, 0)
        return 0
    lax.fori_loop(0, nb, _b, 0)

    def _f(i, _):
        r01 = plsc.load_gather(rp, [i + iota])

        @pl.when(r01[1] == r01[0])
        def _():
            for c in range(8):
                out_v[i, pl.ds(c * 16, 16)] = cs_v[pl.ds(c * 16, 16)] * (1.0 / N)
        return 0
    lax.fori_loop(0, NPW, _f, 0)

    pltpu.sync_copy(out_v, out_hbm.at[pl.ds(lo, NPW)])


def _pb_call(abuf, acnt, s, t, wh, cs):
    mesh = plsc.VectorSubcoreMesh(core_axis_name="c", subcore_axis_name="s")
    f = pl.kernel(
        _pb_body,
        out_type=jax.ShapeDtypeStruct((NPAD, D), jnp.float32),
        mesh=mesh,
        compiler_params=pltpu.CompilerParams(needs_layout_passes=False),
        scratch_types=[
            pltpu.VMEM((N,), jnp.float32),
            pltpu.VMEM((N,), jnp.float32),
            pltpu.VMEM((D,), jnp.float32),
            pltpu.VMEM((NW * 32,), jnp.int32),
            pltpu.VMEM((ABUF,), jnp.int32),
            pltpu.VMEM((SZ,), jnp.int32),
            pltpu.VMEM((SZ,), jnp.int32),
            pltpu.VMEM((SZ,), jnp.float32),
            pltpu.VMEM((SZ,), jnp.float32),
            pltpu.VMEM((NPW + 16,), jnp.int32),
            pltpu.VMEM((NPW + 16,), jnp.int32),
            pltpu.VMEM((NPW + 16,), jnp.int32),
            pltpu.VMEM((NPW,), jnp.float32),
            pltpu.VMEM((NPW,), jnp.float32),
            pltpu.VMEM((N,), jnp.int32),
            pltpu.VMEM((BR,), jnp.int32),
            pltpu.VMEM((BR, D), jnp.float32),
            pltpu.VMEM((NPW, D), jnp.float32),
            pltpu.SemaphoreType.DMA,
        ],
    )
    return f(abuf, acnt, s, t, wh, cs)


@jax.jit
def kernel(X, edges, W, a):
    A2 = jnp.concatenate([a[:D], a[D:]], axis=1)
    wh, st, cs = _tc_call(X, W, A2)
    abuf, acnt = _pa_call(edges[0], edges[1])
    out = _pb_call(abuf, acnt, st[:, 0], st[:, 1], wh, cs.reshape(D))
    return out[:N]


# speedup vs baseline: 1.1598x; 1.1598x over previous
import jax
import jax.numpy as jnp
from jax import lax
from jax.experimental import pallas as pl
from jax.experimental.pallas import tpu as pltpu
from jax.experimental.pallas import tpu_sc as plsc

N = 10000
E = 160000
D = 128
ALPHA = 0.2

NW = 32
NPW = 320
NPAD = NW * NPW
EPW = E // NW
BCAP = 288
ABUF = NW * BCAP
CAP = 5888
SZ = CAP + 64
BR = 64
NEG = -3.4e38
HUGE = 0x7FFFFFFF


def _tc_body(x_ref, w_ref, a2_ref, wh_ref, st_ref, cs_ref):
    i = pl.program_id(0)
    wh = jnp.dot(x_ref[...], w_ref[...], preferred_element_type=jnp.float32)
    wh_ref[...] = wh
    st_ref[...] = jnp.dot(wh, a2_ref[...], preferred_element_type=jnp.float32)

    @pl.when(i == 0)
    def _():
        cs_ref[...] = jnp.zeros_like(cs_ref)

    cs_ref[...] += jnp.sum(wh, axis=0, keepdims=True)


def _tc_call(X, W, A2):
    return pl.pallas_call(
        _tc_body,
        grid=(10,),
        in_specs=[
            pl.BlockSpec((1000, D), lambda i: (i, 0)),
            pl.BlockSpec((D, D), lambda i: (0, 0)),
            pl.BlockSpec((D, 2), lambda i: (0, 0)),
        ],
        out_specs=[
            pl.BlockSpec((1000, D), lambda i: (i, 0)),
            pl.BlockSpec((1000, 2), lambda i: (i, 0)),
            pl.BlockSpec((1, D), lambda i: (0, 0)),
        ],
        out_shape=[
            jax.ShapeDtypeStruct((N, D), jnp.float32),
            jax.ShapeDtypeStruct((N, 2), jnp.float32),
            jax.ShapeDtypeStruct((1, D), jnp.float32),
        ],
    )(X, W, A2)


def _pa_body(src_hbm, dst_hbm, abuf_hbm, acnt_hbm, src_v, dst_v, ab_v, ac_v):
    wid = lax.axis_index("c") * 16 + lax.axis_index("s")
    iota = lax.iota(jnp.int32, 16)
    zi = jnp.zeros((16,), jnp.int32)

    pltpu.sync_copy(src_hbm.at[pl.ds(wid * EPW, EPW)], src_v.at[pl.ds(0, EPW)])
    pltpu.sync_copy(dst_hbm.at[pl.ds(wid * EPW, EPW)], dst_v.at[pl.ds(0, EPW)])

    for q in range(3):
        ac_v[pl.ds(q * 16, 16)] = zi

    def _scan(j, _):
        sv = src_v[pl.ds(j * 16, 16)]
        dv = dst_v[pl.ds(j * 16, 16)]
        valid = j * 16 + iota < EPW
        key = jnp.where(valid, sv * 16384 + dv, jnp.int32(HUGE))
        sk, _si = plsc.sort_key_val(key, iota)
        vs = sk < HUGE
        own = jnp.where(vs, ((sk >> 14) * 6554) >> 21, jnp.int32(32))
        prev = own[jnp.clip(iota - 1, 0, 15)]
        neq = (own != prev) | (iota == 0)
        start = plsc.cummax(jnp.where(neq, iota, 0))
        rank = iota - start
        base = plsc.load_gather(ac_v, [own])
        slot = jnp.minimum(base + rank, BCAP - 1)
        pos = jnp.minimum(own, NW - 1) * BCAP + slot
        plsc.store_scatter(ab_v, [pos], sk, mask=vs)
        nxt = own[jnp.clip(iota + 1, 0, 15)]
        is_last = ((own != nxt) | (iota == 15)) & vs
        plsc.store_scatter(ac_v, [own], jnp.minimum(slot + 1, BCAP),
                           mask=is_last)
        return 0

    lax.fori_loop(0, (EPW + 15) // 16, _scan, 0)

    pltpu.sync_copy(ab_v, abuf_hbm.at[pl.ds(wid * ABUF, ABUF)])
    pltpu.sync_copy(ac_v.at[pl.ds(0, 32)], acnt_hbm.at[pl.ds(wid * 32, 32)])


def _pa_call(src, dst):
    mesh = plsc.VectorSubcoreMesh(core_axis_name="c", subcore_axis_name="s")
    f = pl.kernel(
        _pa_body,
        out_type=[
            jax.ShapeDtypeStruct((NW * ABUF,), jnp.int32),
            jax.ShapeDtypeStruct((NW * 32,), jnp.int32),
        ],
        mesh=mesh,
        compiler_params=pltpu.CompilerParams(needs_layout_passes=False),
        scratch_types=[
            pltpu.VMEM((EPW + 16,), jnp.int32),
            pltpu.VMEM((EPW + 16,), jnp.int32),
            pltpu.VMEM((ABUF,), jnp.int32),
            pltpu.VMEM((48,), jnp.int32),
        ],
    )
    return f(src, dst)


def _pb_body(abuf_hbm, acnt_hbm, s_hbm, t_hbm, wh_hbm, cs_hbm, out_hbm,
             s_v, t_v, cs_v, ac_all, bb_v, src_s, dst_s, e_s, att_s,
             hist, rp, cur, m_arr, rz_arr, stamp, dstb, dstb2, gbuf, gbuf2,
             out_v, sem, sem2):
    wid = lax.axis_index("c") * 16 + lax.axis_index("s")
    lo = wid * NPW
    iota = lax.iota(jnp.int32, 16)
    zf = jnp.zeros((16,), jnp.float32)
    zi = jnp.zeros((16,), jnp.int32)

    def spl(x):
        return jnp.full((16,), x)

    def g1(ref, i):
        return plsc.load_gather(ref, [spl(i)])

    def s1(ref, i, x):
        xv = x if getattr(x, "shape", ()) == (16,) else spl(x)
        plsc.store_scatter(ref, [spl(i)], xv)

    pltpu.sync_copy(s_hbm, s_v)
    pltpu.sync_copy(t_hbm, t_v)
    pltpu.sync_copy(cs_hbm, cs_v)
    pltpu.sync_copy(acnt_hbm, ac_all)

    def _ld(s2, _):
        pltpu.sync_copy(abuf_hbm.at[pl.ds(s2 * ABUF + wid * BCAP, BCAP)],
                        bb_v.at[pl.ds(s2 * BCAP, BCAP)])
        return 0
    lax.fori_loop(0, NW, _ld, 0)

    def _mz(j, _):
        hist[pl.ds(j * 16, 16)] = zi
        return 0
    lax.fori_loop(0, (NPW + 16) // 16, _mz, 0)

    def _ms(j, _):
        stamp[pl.ds(j * 16, 16)] = zi - 1
        return 0
    lax.fori_loop(0, N // 16, _ms, 0)

    def _md(j, _):
        dst_s[pl.ds(j * 16, 16)] = zi
        return 0
    lax.fori_loop(0, SZ // 16, _md, 0)

    def _mo(i, _):
        for c in range(8):
            out_v[i, pl.ds(c * 16, 16)] = zf
        return 0
    lax.fori_loop(0, NPW, _mo, 0)

    def _h_seg(s2, _):
        cnt_s = g1(ac_all, s2 * 32 + wid)[0]

        def _h(j, _):
            kv = bb_v[pl.ds(s2 * BCAP + j * 16, 16)]
            key = jnp.where(j * 16 + iota < cnt_s, kv, jnp.int32(HUGE))
            sk, _si = plsc.sort_key_val(key, iota)
            vs = sk < HUGE
            row = jnp.where(vs, (sk >> 14) - lo, jnp.int32(NPW))
            prev = row[jnp.clip(iota - 1, 0, 15)]
            neq = (row != prev) | (iota == 0)
            start = plsc.cummax(jnp.where(neq, iota, 0))
            rank = iota - start
            base = plsc.load_gather(hist, [row])
            nxt = row[jnp.clip(iota + 1, 0, 15)]
            is_last = (row != nxt) | (iota == 15)
            plsc.store_scatter(hist, [row], base + rank + 1, mask=is_last)
            return 0

        lax.fori_loop(0, (cnt_s + 15) // 16, _h, 0)
        return 0
    lax.fori_loop(0, NW, _h_seg, 0)

    def _p(k, acc):
        v = hist[pl.ds(k * 16, 16)]
        c = plsc.cumsum(v)
        rp[pl.ds(k * 16, 16)] = spl(acc) + c - v
        cur[pl.ds(k * 16, 16)] = spl(acc) + c - v
        return acc + c[15]
    cnt = lax.fori_loop(0, NPW // 16, _p, jnp.int32(0))
    s1(rp, NPW, cnt)

    def _pl_seg(s2, _):
        cnt_s = g1(ac_all, s2 * 32 + wid)[0]

        def _pl2(j, _):
            kv = bb_v[pl.ds(s2 * BCAP + j * 16, 16)]
            key = jnp.where(j * 16 + iota < cnt_s, kv, jnp.int32(HUGE))
            sk, _si = plsc.sort_key_val(key, iota)
            vs = sk < HUGE
            row = jnp.where(vs, (sk >> 14) - lo, jnp.int32(NPW))
            prev = row[jnp.clip(iota - 1, 0, 15)]
            neq = (row != prev) | (iota == 0)
            start = plsc.cummax(jnp.where(neq, iota, 0))
            rank = iota - start
            base = plsc.load_gather(cur, [row])
            pos = jnp.clip(base + rank, 0, SZ - 1)
            plsc.store_scatter(src_s, [pos], sk >> 14, mask=vs)
            plsc.store_scatter(dst_s, [pos], sk & 16383, mask=vs)
            nxt = row[jnp.clip(iota + 1, 0, 15)]
            is_last = (row != nxt) | (iota == 15)
            plsc.store_scatter(cur, [row], base + rank + 1, mask=is_last)
            return 0

        lax.fori_loop(0, (cnt_s + 15) // 16, _pl2, 0)
        return 0
    lax.fori_loop(0, NW, _pl_seg, 0)

    nv = ((cnt + BR - 1) // BR) * (BR // 16)

    def _e(j, _):
        idxc = jnp.minimum(j * 16 + iota, cnt - 1)
        sv = plsc.load_gather(src_s, [idxc])
        dv = plsc.load_gather(dst_s, [idxc])
        z = plsc.load_gather(s_v, [sv]) + plsc.load_gather(t_v, [dv])
        e_s[pl.ds(j * 16, 16)] = jnp.where(z > 0, z, ALPHA * z)
        return 0
    lax.fori_loop(0, nv, _e, 0)

    negv = jnp.full((16,), NEG, jnp.float32)

    def _d(j, _):
        idxv = j * 16 + iota
        idxc = jnp.minimum(idxv, cnt - 1)
        valid = idxv < cnt
        sv = plsc.load_gather(src_s, [idxc])
        dv = plsc.load_gather(dst_s, [idxc])
        sl = jnp.clip(sv - lo, 0, NPW - 1)
        key2 = jnp.where(valid, dv * 512 + sl, jnp.int32(HUGE))
        sk, si = plsc.sort_key_val(key2, iota)
        gidx = j * 16 + si
        prev = sk[jnp.clip(iota - 1, 0, 15)]
        eqprev = (sk == prev) & (iota > 0) & (sk < HUGE)
        plsc.store_scatter(e_s, [jnp.clip(gidx, 0, SZ - 1)], negv,
                           mask=eqprev)
        rp0 = plsc.load_gather(rp, [sl])
        old = plsc.load_gather(stamp, [dv])
        dupb = valid & (old >= rp0)
        plsc.store_scatter(e_s, [idxc], negv, mask=dupb)
        nxt = sk[jnp.clip(iota + 1, 0, 15)]
        is_last = (((sk >> 9) != (nxt >> 9)) | (iota == 15)) & (sk < HUGE)
        plsc.store_scatter(stamp, [jnp.clip(sk >> 9, 0, N - 1)], gidx,
                           mask=is_last)
        return 0
    lax.fori_loop(0, (cnt + 15) // 16, _d, 0)

    def _r(i, _):
        r01 = plsc.load_gather(rp, [i + iota])
        r0 = r01[0]
        r1 = r01[1]

        @pl.when(r1 > r0)
        def _():
            nch = (r1 - r0 + 15) // 16

            def _m(c2, m):
                idxv = r0 + c2 * 16 + iota
                ev = plsc.load_gather(e_s, [jnp.minimum(idxv, r1 - 1)])
                return jnp.maximum(m, jnp.max(jnp.where(idxv < r1, ev, NEG)))

            m = lax.fori_loop(0, nch, _m, jnp.float32(NEG))

            def _z(c2, za):
                idxv = r0 + c2 * 16 + iota
                ev = plsc.load_gather(e_s, [jnp.minimum(idxv, r1 - 1)])
                return za + jnp.sum(jnp.where(idxv < r1, jnp.exp(ev - m), 0.0))

            zs = lax.fori_loop(0, nch, _z, jnp.float32(0.0))
            s1(m_arr, i, m)
            s1(rz_arr, i, 1.0 / spl(zs))
        return 0
    lax.fori_loop(0, NPW, _r, 0)

    def _a(j, _):
        idxv = j * 16 + iota
        idxc = jnp.minimum(idxv, cnt - 1)
        sv = plsc.load_gather(src_s, [idxc])
        sl = jnp.clip(sv - lo, 0, NPW - 1)
        mrow = plsc.load_gather(m_arr, [sl])
        rz = plsc.load_gather(rz_arr, [sl])
        ev = plsc.load_gather(e_s, [idxc])
        att = jnp.where(idxv < cnt, jnp.exp(ev - mrow) * rz, 0.0)
        att_s[pl.ds(j * 16, 16)] = att
        return 0
    lax.fori_loop(0, nv, _a, 0)

    nb = (cnt + BR - 1) // BR

    def _fill(b, dref):
        base = b * BR
        for q in range(BR // 16):
            dref[pl.ds(q * 16, 16)] = dst_s[pl.ds(base + q * 16, 16)]

    @pl.when(nb > 0)
    def _():
        _fill(0, dstb)
        pltpu.async_copy(wh_hbm.at[dstb], gbuf, sem)

    def _accum_from(gb, base):
        def _acc(l, _):
            r = base + l
            al = g1(att_s, r)
            sl = jnp.clip(g1(src_s, r)[0] - lo, 0, NPW - 1)
            for c in range(8):
                seg = pl.ds(c * 16, 16)
                out_v[sl, seg] = out_v[sl, seg] + al * gb[l, seg]
            return 0
        lax.fori_loop(0, BR, _acc, 0)

    def _b(b, _):
        @pl.when(b % 2 == 0)
        def _():
            pltpu.make_async_copy(wh_hbm.at[dstb], gbuf, sem).wait()

            @pl.when(b + 1 < nb)
            def _():
                _fill(b + 1, dstb2)
                pltpu.async_copy(wh_hbm.at[dstb2], gbuf2, sem2)
            _accum_from(gbuf, b * BR)

        @pl.when(b % 2 == 1)
        def _():
            pltpu.make_async_copy(wh_hbm.at[dstb2], gbuf2, sem2).wait()

            @pl.when(b + 1 < nb)
            def _():
                _fill(b + 1, dstb)
                pltpu.async_copy(wh_hbm.at[dstb], gbuf, sem)
            _accum_from(gbuf2, b * BR)
        return 0
    lax.fori_loop(0, nb, _b, 0)

    def _f(i, _):
        r01 = plsc.load_gather(rp, [i + iota])

        @pl.when(r01[1] == r01[0])
        def _():
            for c in range(8):
                out_v[i, pl.ds(c * 16, 16)] = cs_v[pl.ds(c * 16, 16)] * (1.0 / N)
        return 0
    lax.fori_loop(0, NPW, _f, 0)

    pltpu.sync_copy(out_v, out_hbm.at[pl.ds(lo, NPW)])


def _pb_call(abuf, acnt, s, t, wh, cs):
    mesh = plsc.VectorSubcoreMesh(core_axis_name="c", subcore_axis_name="s")
    f = pl.kernel(
        _pb_body,
        out_type=jax.ShapeDtypeStruct((NPAD, D), jnp.float32),
        mesh=mesh,
        compiler_params=pltpu.CompilerParams(needs_layout_passes=False),
        scratch_types=[
            pltpu.VMEM((N,), jnp.float32),
            pltpu.VMEM((N,), jnp.float32),
            pltpu.VMEM((D,), jnp.float32),
            pltpu.VMEM((NW * 32,), jnp.int32),
            pltpu.VMEM((ABUF,), jnp.int32),
            pltpu.VMEM((SZ,), jnp.int32),
            pltpu.VMEM((SZ,), jnp.int32),
            pltpu.VMEM((SZ,), jnp.float32),
            pltpu.VMEM((SZ,), jnp.float32),
            pltpu.VMEM((NPW + 16,), jnp.int32),
            pltpu.VMEM((NPW + 16,), jnp.int32),
            pltpu.VMEM((NPW + 16,), jnp.int32),
            pltpu.VMEM((NPW,), jnp.float32),
            pltpu.VMEM((NPW,), jnp.float32),
            pltpu.VMEM((N,), jnp.int32),
            pltpu.VMEM((BR,), jnp.int32),
            pltpu.VMEM((BR,), jnp.int32),
            pltpu.VMEM((BR, D), jnp.float32),
            pltpu.VMEM((BR, D), jnp.float32),
            pltpu.VMEM((NPW, D), jnp.float32),
            pltpu.SemaphoreType.DMA,
            pltpu.SemaphoreType.DMA,
        ],
    )
    return f(abuf, acnt, s, t, wh, cs)


@jax.jit
def kernel(X, edges, W, a):
    A2 = jnp.concatenate([a[:D], a[D:]], axis=1)
    wh, st, cs = _tc_call(X, W, A2)
    abuf, acnt = _pa_call(edges[0], edges[1])
    out = _pb_call(abuf, acnt, st[:, 0], st[:, 1], wh, cs.reshape(D))
    return out[:N]


# speedup vs baseline: 1.5893x; 1.3703x over previous
import jax
import jax.numpy as jnp
from jax import lax
from jax.experimental import pallas as pl
from jax.experimental.pallas import tpu as pltpu
from jax.experimental.pallas import tpu_sc as plsc

N = 10000
E = 160000
D = 128
ALPHA = 0.2

NW = 32
NPW = 320
NPAD = NW * NPW
EPW = E // NW
BCAP = 288
ABUF = NW * BCAP
CAP = 5888
SZ = CAP + 128
BR = 64
NEG = -3.4e38
HUGE = 0x7FFFFFFF


def _tc_body(x_ref, w_ref, a2_ref, wh_ref, st_ref, cs_ref):
    i = pl.program_id(0)
    wh = jnp.dot(x_ref[...], w_ref[...], preferred_element_type=jnp.float32)
    wh_ref[...] = wh
    st_ref[...] = jnp.dot(wh, a2_ref[...], preferred_element_type=jnp.float32)

    @pl.when(i == 0)
    def _():
        cs_ref[...] = jnp.zeros_like(cs_ref)

    cs_ref[...] += jnp.sum(wh, axis=0, keepdims=True)


def _tc_call(X, W, A2):
    return pl.pallas_call(
        _tc_body,
        grid=(10,),
        in_specs=[
            pl.BlockSpec((1000, D), lambda i: (i, 0)),
            pl.BlockSpec((D, D), lambda i: (0, 0)),
            pl.BlockSpec((D, 2), lambda i: (0, 0)),
        ],
        out_specs=[
            pl.BlockSpec((1000, D), lambda i: (i, 0)),
            pl.BlockSpec((1000, 2), lambda i: (i, 0)),
            pl.BlockSpec((1, D), lambda i: (0, 0)),
        ],
        out_shape=[
            jax.ShapeDtypeStruct((N, D), jnp.float32),
            jax.ShapeDtypeStruct((N, 2), jnp.float32),
            jax.ShapeDtypeStruct((1, D), jnp.float32),
        ],
    )(X, W, A2)


def _pa_body(src_hbm, dst_hbm, abuf_hbm, acnt_hbm, src_v, dst_v, ab_v, ac_v):
    wid = lax.axis_index("c") * 16 + lax.axis_index("s")
    iota = lax.iota(jnp.int32, 16)
    zi = jnp.zeros((16,), jnp.int32)

    pltpu.sync_copy(src_hbm.at[pl.ds(wid * EPW, EPW)], src_v.at[pl.ds(0, EPW)])
    pltpu.sync_copy(dst_hbm.at[pl.ds(wid * EPW, EPW)], dst_v.at[pl.ds(0, EPW)])

    for q in range(3):
        ac_v[pl.ds(q * 16, 16)] = zi

    def _scan(j, _):
        sv = src_v[pl.ds(j * 16, 16)]
        dv = dst_v[pl.ds(j * 16, 16)]
        valid = j * 16 + iota < EPW
        key = jnp.where(valid, sv * 16384 + dv, jnp.int32(HUGE))
        sk, _si = plsc.sort_key_val(key, iota)
        vs = sk < HUGE
        own = jnp.where(vs, ((sk >> 14) * 6554) >> 21, jnp.int32(32))
        prev = own[jnp.clip(iota - 1, 0, 15)]
        neq = (own != prev) | (iota == 0)
        start = plsc.cummax(jnp.where(neq, iota, 0))
        rank = iota - start
        base = plsc.load_gather(ac_v, [own])
        slot = jnp.minimum(base + rank, BCAP - 1)
        pos = jnp.minimum(own, NW - 1) * BCAP + slot
        plsc.store_scatter(ab_v, [pos], sk, mask=vs)
        nxt = own[jnp.clip(iota + 1, 0, 15)]
        is_last = ((own != nxt) | (iota == 15)) & vs
        plsc.store_scatter(ac_v, [own], jnp.minimum(slot + 1, BCAP),
                           mask=is_last)
        return 0

    lax.fori_loop(0, (EPW + 15) // 16, _scan, 0)

    pltpu.sync_copy(ab_v, abuf_hbm.at[pl.ds(wid * ABUF, ABUF)])
    pltpu.sync_copy(ac_v.at[pl.ds(0, 32)], acnt_hbm.at[pl.ds(wid * 32, 32)])


def _pa_call(src, dst):
    mesh = plsc.VectorSubcoreMesh(core_axis_name="c", subcore_axis_name="s")
    f = pl.kernel(
        _pa_body,
        out_type=[
            jax.ShapeDtypeStruct((NW * ABUF,), jnp.int32),
            jax.ShapeDtypeStruct((NW * 32,), jnp.int32),
        ],
        mesh=mesh,
        compiler_params=pltpu.CompilerParams(needs_layout_passes=False),
        scratch_types=[
            pltpu.VMEM((EPW + 16,), jnp.int32),
            pltpu.VMEM((EPW + 16,), jnp.int32),
            pltpu.VMEM((ABUF,), jnp.int32),
            pltpu.VMEM((48,), jnp.int32),
        ],
    )
    return f(src, dst)


def _pb_body(abuf_hbm, acnt_hbm, s_hbm, t_hbm, wh_hbm, cs_hbm, out_hbm,
             s_v, t_v, cs_v, ac_all, bb_v, src_s, dst_s, e_s, att_s,
             hist, rp, cur, m_arr, rz_arr, stamp, dstb, dstb2, gbuf, gbuf2,
             out_v, sem, sem2, sem3):
    wid = lax.axis_index("c") * 16 + lax.axis_index("s")
    lo = wid * NPW
    iota = lax.iota(jnp.int32, 16)
    zf = jnp.zeros((16,), jnp.float32)
    zi = jnp.zeros((16,), jnp.int32)

    def spl(x):
        return jnp.full((16,), x)

    def g1(ref, i):
        return plsc.load_gather(ref, [spl(i)])

    def s1(ref, i, x):
        xv = x if getattr(x, "shape", ()) == (16,) else spl(x)
        plsc.store_scatter(ref, [spl(i)], xv)

    pltpu.sync_copy(s_hbm, s_v)
    pltpu.sync_copy(t_hbm, t_v)
    pltpu.sync_copy(cs_hbm, cs_v)
    pltpu.sync_copy(acnt_hbm, ac_all)

    def _ld(s2, _):
        pltpu.async_copy(abuf_hbm.at[pl.ds(s2 * ABUF + wid * BCAP, BCAP)],
                         bb_v.at[pl.ds(s2 * BCAP, BCAP)], sem3)
        return 0
    lax.fori_loop(0, NW, _ld, 0)

    def _mz(j, _):
        hist[pl.ds(j * 16, 16)] = zi
        return 0
    lax.fori_loop(0, (NPW + 16) // 16, _mz, 0)

    def _ms(j, _):
        stamp[pl.ds(j * 16, 16)] = zi - 1
        return 0
    lax.fori_loop(0, N // 16, _ms, 0)

    def _md(j, _):
        dst_s[pl.ds(j * 16, 16)] = zi
        return 0
    lax.fori_loop(0, SZ // 16, _md, 0)

    def _mo(i, _):
        for c in range(8):
            out_v[i, pl.ds(c * 16, 16)] = zf
        return 0
    lax.fori_loop(0, NPW, _mo, 0)

    def _ldw(s2, _):
        pltpu.make_async_copy(
            abuf_hbm.at[pl.ds(s2 * ABUF + wid * BCAP, BCAP)],
            bb_v.at[pl.ds(s2 * BCAP, BCAP)], sem3).wait()
        return 0
    lax.fori_loop(0, NW, _ldw, 0)

    def _h_seg(s2, _):
        cnt_s = g1(ac_all, s2 * 32 + wid)[0]

        def _h(j, _):
            kv = bb_v[pl.ds(s2 * BCAP + j * 16, 16)]
            key = jnp.where(j * 16 + iota < cnt_s, kv, jnp.int32(HUGE))
            sk, _si = plsc.sort_key_val(key, iota)
            vs = sk < HUGE
            row = jnp.where(vs, (sk >> 14) - lo, jnp.int32(NPW))
            prev = row[jnp.clip(iota - 1, 0, 15)]
            neq = (row != prev) | (iota == 0)
            start = plsc.cummax(jnp.where(neq, iota, 0))
            rank = iota - start
            base = plsc.load_gather(hist, [row])
            nxt = row[jnp.clip(iota + 1, 0, 15)]
            is_last = (row != nxt) | (iota == 15)
            plsc.store_scatter(hist, [row], base + rank + 1, mask=is_last)
            return 0

        lax.fori_loop(0, (cnt_s + 15) // 16, _h, 0)
        return 0
    lax.fori_loop(0, NW, _h_seg, 0)

    def _p(k, acc):
        v = hist[pl.ds(k * 16, 16)]
        c = plsc.cumsum(v)
        rp[pl.ds(k * 16, 16)] = spl(acc) + c - v
        cur[pl.ds(k * 16, 16)] = spl(acc) + c - v
        return acc + c[15]
    cnt = lax.fori_loop(0, NPW // 16, _p, jnp.int32(0))
    s1(rp, NPW, cnt)

    def _pl_seg(s2, _):
        cnt_s = g1(ac_all, s2 * 32 + wid)[0]

        def _pl2(j, _):
            kv = bb_v[pl.ds(s2 * BCAP + j * 16, 16)]
            key = jnp.where(j * 16 + iota < cnt_s, kv, jnp.int32(HUGE))
            sk, _si = plsc.sort_key_val(key, iota)
            vs = sk < HUGE
            row = jnp.where(vs, (sk >> 14) - lo, jnp.int32(NPW))
            prev = row[jnp.clip(iota - 1, 0, 15)]
            neq = (row != prev) | (iota == 0)
            start = plsc.cummax(jnp.where(neq, iota, 0))
            rank = iota - start
            base = plsc.load_gather(cur, [row])
            pos = jnp.clip(base + rank, 0, SZ - 1)
            plsc.store_scatter(src_s, [pos], sk >> 14, mask=vs)
            plsc.store_scatter(dst_s, [pos], sk & 16383, mask=vs)
            nxt = row[jnp.clip(iota + 1, 0, 15)]
            is_last = (row != nxt) | (iota == 15)
            plsc.store_scatter(cur, [row], base + rank + 1, mask=is_last)
            return 0

        lax.fori_loop(0, (cnt_s + 15) // 16, _pl2, 0)
        return 0
    lax.fori_loop(0, NW, _pl_seg, 0)

    nv = ((cnt + 2 * BR - 1) // (2 * BR)) * (2 * BR // 16)

    def _e(j, _):
        idxc = jnp.minimum(j * 16 + iota, cnt - 1)
        sv = plsc.load_gather(src_s, [idxc])
        dv = plsc.load_gather(dst_s, [idxc])
        z = plsc.load_gather(s_v, [sv]) + plsc.load_gather(t_v, [dv])
        e_s[pl.ds(j * 16, 16)] = jnp.where(z > 0, z, ALPHA * z)
        return 0
    lax.fori_loop(0, nv, _e, 0)

    negv = jnp.full((16,), NEG, jnp.float32)

    def _d(j, _):
        idxv = j * 16 + iota
        idxc = jnp.minimum(idxv, cnt - 1)
        valid = idxv < cnt
        sv = plsc.load_gather(src_s, [idxc])
        dv = plsc.load_gather(dst_s, [idxc])
        sl = jnp.clip(sv - lo, 0, NPW - 1)
        key2 = jnp.where(valid, dv * 512 + sl, jnp.int32(HUGE))
        sk, si = plsc.sort_key_val(key2, iota)
        gidx = j * 16 + si
        prev = sk[jnp.clip(iota - 1, 0, 15)]
        eqprev = (sk == prev) & (iota > 0) & (sk < HUGE)
        plsc.store_scatter(e_s, [jnp.clip(gidx, 0, SZ - 1)], negv,
                           mask=eqprev)
        rp0 = plsc.load_gather(rp, [sl])
        old = plsc.load_gather(stamp, [dv])
        dupb = valid & (old >= rp0)
        plsc.store_scatter(e_s, [idxc], negv, mask=dupb)
        nxt = sk[jnp.clip(iota + 1, 0, 15)]
        is_last = (((sk >> 9) != (nxt >> 9)) | (iota == 15)) & (sk < HUGE)
        plsc.store_scatter(stamp, [jnp.clip(sk >> 9, 0, N - 1)], gidx,
                           mask=is_last)
        return 0
    lax.fori_loop(0, (cnt + 15) // 16, _d, 0)

    def _r(i, _):
        r01 = plsc.load_gather(rp, [i + iota])
        r0 = r01[0]
        r1 = r01[1]

        @pl.when(r1 > r0)
        def _():
            nch = (r1 - r0 + 15) // 16

            def _m(c2, m):
                idxv = r0 + c2 * 16 + iota
                ev = plsc.load_gather(e_s, [jnp.minimum(idxv, r1 - 1)])
                return jnp.maximum(m, jnp.max(jnp.where(idxv < r1, ev, NEG)))

            m = lax.fori_loop(0, nch, _m, jnp.float32(NEG))

            def _z(c2, za):
                idxv = r0 + c2 * 16 + iota
                ev = plsc.load_gather(e_s, [jnp.minimum(idxv, r1 - 1)])
                return za + jnp.sum(jnp.where(idxv < r1, jnp.exp(ev - m), 0.0))

            zs = lax.fori_loop(0, nch, _z, jnp.float32(0.0))
            s1(m_arr, i, m)
            s1(rz_arr, i, 1.0 / spl(zs))
        return 0
    lax.fori_loop(0, NPW, _r, 0)

    def _a(j, _):
        idxv = j * 16 + iota
        idxc = jnp.minimum(idxv, cnt - 1)
        sv = plsc.load_gather(src_s, [idxc])
        sl = jnp.clip(sv - lo, 0, NPW - 1)
        mrow = plsc.load_gather(m_arr, [sl])
        rz = plsc.load_gather(rz_arr, [sl])
        ev = plsc.load_gather(e_s, [idxc])
        att = jnp.where(idxv < cnt, jnp.exp(ev - mrow) * rz, 0.0)
        att_s[pl.ds(j * 16, 16)] = att
        return 0
    lax.fori_loop(0, nv, _a, 0)

    nb2 = ((cnt + 2 * BR - 1) // (2 * BR)) * 2

    def _fill(b, dref):
        base = b * BR
        for q in range(BR // 16):
            dref[pl.ds(q * 16, 16)] = dst_s[pl.ds(base + q * 16, 16)]

    @pl.when(nb2 > 0)
    def _():
        _fill(0, dstb)
        pltpu.async_copy(wh_hbm.at[dstb], gbuf, sem)

    def _accum_from(gb, base, carry):
        def _acc(l, c9):
            r = base + l
            al = g1(att_s, r)
            sl = jnp.clip(g1(src_s, r)[0] - lo, 0, NPW - 1)
            cur_row = c9[8]
            is_new = sl != cur_row

            @pl.when(is_new)
            def _():
                for c in range(8):
                    seg = pl.ds(c * 16, 16)
                    out_v[cur_row, seg] = out_v[cur_row, seg] + c9[c]

            keep = jnp.where(is_new, 0.0, 1.0)
            acc = tuple(c9[c] * keep + al * gb[l, pl.ds(c * 16, 16)]
                        for c in range(8))
            return acc + (sl,)
        return lax.fori_loop(0, BR, _acc, carry)

    def _b(p, carry):
        b0 = 2 * p
        pltpu.make_async_copy(wh_hbm.at[dstb], gbuf, sem).wait()
        _fill(b0 + 1, dstb2)
        pltpu.async_copy(wh_hbm.at[dstb2], gbuf2, sem2)
        carry = _accum_from(gbuf, b0 * BR, carry)
        pltpu.make_async_copy(wh_hbm.at[dstb2], gbuf2, sem2).wait()

        @pl.when(b0 + 2 < nb2)
        def _():
            _fill(b0 + 2, dstb)
            pltpu.async_copy(wh_hbm.at[dstb], gbuf, sem)
        return _accum_from(gbuf2, (b0 + 1) * BR, carry)

    carry0 = tuple(zf for _ in range(8)) + (jnp.int32(0),)
    carry = lax.fori_loop(0, nb2 // 2, _b, carry0)
    last_row = carry[8]
    for c in range(8):
        seg = pl.ds(c * 16, 16)
        out_v[last_row, seg] = out_v[last_row, seg] + carry[c]

    def _f(i, _):
        r01 = plsc.load_gather(rp, [i + iota])

        @pl.when(r01[1] == r01[0])
        def _():
            for c in range(8):
                out_v[i, pl.ds(c * 16, 16)] = cs_v[pl.ds(c * 16, 16)] * (1.0 / N)
        return 0
    lax.fori_loop(0, NPW, _f, 0)

    pltpu.sync_copy(out_v, out_hbm.at[pl.ds(lo, NPW)])


def _pb_call(abuf, acnt, s, t, wh, cs):
    mesh = plsc.VectorSubcoreMesh(core_axis_name="c", subcore_axis_name="s")
    f = pl.kernel(
        _pb_body,
        out_type=jax.ShapeDtypeStruct((NPAD, D), jnp.float32),
        mesh=mesh,
        compiler_params=pltpu.CompilerParams(needs_layout_passes=False),
        scratch_types=[
            pltpu.VMEM((N,), jnp.float32),
            pltpu.VMEM((N,), jnp.float32),
            pltpu.VMEM((D,), jnp.float32),
            pltpu.VMEM((NW * 32,), jnp.int32),
            pltpu.VMEM((ABUF,), jnp.int32),
            pltpu.VMEM((SZ,), jnp.int32),
            pltpu.VMEM((SZ,), jnp.int32),
            pltpu.VMEM((SZ,), jnp.float32),
            pltpu.VMEM((SZ,), jnp.float32),
            pltpu.VMEM((NPW + 16,), jnp.int32),
            pltpu.VMEM((NPW + 16,), jnp.int32),
            pltpu.VMEM((NPW + 16,), jnp.int32),
            pltpu.VMEM((NPW,), jnp.float32),
            pltpu.VMEM((NPW,), jnp.float32),
            pltpu.VMEM((N,), jnp.int32),
            pltpu.VMEM((BR,), jnp.int32),
            pltpu.VMEM((BR,), jnp.int32),
            pltpu.VMEM((BR, D), jnp.float32),
            pltpu.VMEM((BR, D), jnp.float32),
            pltpu.VMEM((NPW, D), jnp.float32),
            pltpu.SemaphoreType.DMA,
            pltpu.SemaphoreType.DMA,
            pltpu.SemaphoreType.DMA,
        ],
    )
    return f(abuf, acnt, s, t, wh, cs)


@jax.jit
def kernel(X, edges, W, a):
    A2 = jnp.concatenate([a[:D], a[D:]], axis=1)
    wh, st, cs = _tc_call(X, W, A2)
    abuf, acnt = _pa_call(edges[0], edges[1])
    out = _pb_call(abuf, acnt, st[:, 0], st[:, 1], wh, cs.reshape(D))
    return out[:N]
